# baseline (speedup 1.0000x reference)
import functools
import math

import jax
import jax.numpy as jnp
from jax import lax
from jax.experimental import pallas as pl
from jax.experimental.pallas import tpu as pltpu
from jax.experimental.pallas import tpu_sc as plsc

N_NODES = 10000
N_EDGES = 10000
NNZ = 320000
CH = 128

NC = 2
NS = 16
NW = NC * NS
CHUNK = 128
C0 = 160
C1 = 0
PH0 = (40, 40, 40, 40)
PH1 = ()
IDXBUF = 40
TOTAL_CHUNKS = NS * (C0 + C1)
NNZ_PAD = TOTAL_CHUNKS * CHUNK
ROWS = 10112
ROWS_PER_TILE = ROWS // NS
TRASH = 10000


def _lorentz_body(x_ref, w_ref, esc_ref, o_ref):
    x = x_ref[...]
    w = w_ref[...]
    y = jnp.dot(x, w.T, preferred_element_type=jnp.float32)
    y0 = y[:, 0:1]
    time = jax.nn.sigmoid(y0) * esc_ref[0, 0] + 1.1
    sq = jnp.sum(y * y, axis=1, keepdims=True) - y0 * y0
    sq = jnp.clip(sq, 1e-8, None)
    s = (time * time - 1.0) / sq
    col = lax.broadcasted_iota(jnp.int32, y.shape, 1)
    o_ref[...] = jnp.where(col == 0, time, y * jnp.sqrt(s))


def _lorentz_tc(X, W, escale):
    grid = 10
    rows = N_NODES // grid
    return pl.pallas_call(
        _lorentz_body,
        grid=(grid,),
        in_specs=[
            pl.BlockSpec((rows, CH), lambda i: (i, 0)),
            pl.BlockSpec((CH, CH), lambda i: (0, 0)),
            pl.BlockSpec(memory_space=pltpu.SMEM),
        ],
        out_specs=pl.BlockSpec((rows, CH), lambda i: (i, 0)),
        out_shape=jax.ShapeDtypeStruct((N_NODES, CH), jnp.float32),
    )(X, W, escale)


def _norm_body(p_ref, o_ref):
    xe = p_ref[0] + p_ref[1]
    x0 = xe[:, 0:1]
    neg_inner = 2.0 * x0 * x0 - jnp.sum(xe * xe, axis=1, keepdims=True)
    denom = jnp.sqrt(jnp.clip(jnp.abs(neg_inner), 1e-8, None))
    o_ref[...] = xe / denom


def _norm_tc(P):
    grid = 4
    rows = ROWS // grid
    return pl.pallas_call(
        _norm_body,
        grid=(grid,),
        in_specs=[pl.BlockSpec((2, rows, CH), lambda i: (0, i, 0))],
        out_specs=pl.BlockSpec((rows, CH), lambda i: (i, 0)),
        out_shape=jax.ShapeDtypeStruct((ROWS, CH), jnp.float32),
    )(P)


def _final_body(q_ref, x1_ref, eps_ref, o_ref):
    o_ref[...] = eps_ref[0, 0] * (q_ref[0] + q_ref[1]) + x1_ref[...]


def _final_tc(Q, X1, eps):
    grid = 10
    rows = N_NODES // grid
    return pl.pallas_call(
        _final_body,
        grid=(grid,),
        in_specs=[
            pl.BlockSpec((2, rows, CH), lambda i: (0, i, 0)),
            pl.BlockSpec((rows, CH), lambda i: (i, 0)),
            pl.BlockSpec(memory_space=pltpu.SMEM),
        ],
        out_specs=pl.BlockSpec((rows, CH), lambda i: (i, 0)),
        out_shape=jax.ShapeDtypeStruct((N_NODES, CH), jnp.float32),
    )(Q, X1, eps)


def _sc_body(table_hbm, gidx_hbm, sidx_hbm, zeros_hbm, out_hbm,
             gidx_v, sidx_v, buf0, buf1, acc, sem0, sem1):
    cid = lax.axis_index("c")
    sid = lax.axis_index("s")
    r0 = sid * ROWS_PER_TILE
    pltpu.sync_copy(zeros_hbm.at[pl.ds(r0, ROWS_PER_TILE)],
                    acc.at[pl.ds(r0, ROWS_PER_TILE)])
    plsc.subcore_barrier()

    def run_chunks(first_chunk, phase_sizes):
        done = 0
        for pc in phase_sizes:
            pbase = first_chunk + done
            done += pc
            pltpu.sync_copy(gidx_hbm.at[pl.ds(pbase, pc)],
                            gidx_v.at[pl.ds(0, pc)])
            pltpu.sync_copy(sidx_hbm.at[pl.ds(pbase, pc)],
                            sidx_v.at[pl.ds(0, pc)])
            pltpu.async_copy(table_hbm.at[gidx_v.at[0]], buf0, sem0)

            def body(t, carry):
                j = 2 * t
                pltpu.async_copy(table_hbm.at[gidx_v.at[j + 1]], buf1, sem1)
                pltpu.make_async_copy(table_hbm.at[gidx_v.at[j]],
                                      buf0, sem0).wait()
                pltpu.sync_copy(buf0, acc.at[sidx_v.at[j]], add=True)

                @pl.when(j + 2 < pc)
                def _():
                    pltpu.async_copy(table_hbm.at[gidx_v.at[j + 2]],
                                     buf0, sem0)

                pltpu.make_async_copy(table_hbm.at[gidx_v.at[j + 1]],
                                      buf1, sem1).wait()
                pltpu.sync_copy(buf1, acc.at[sidx_v.at[j + 1]], add=True)
                return carry

            lax.fori_loop(0, pc // 2, body, 0)

    @pl.when(cid == 0)
    def _():
        run_chunks(sid * C0, PH0)

    @pl.when(cid == 1)
    def _():
        run_chunks(NS * C0 + sid * C1, PH1)

    plsc.subcore_barrier()
    pltpu.sync_copy(acc.at[pl.ds(r0, ROWS_PER_TILE)],
                    out_hbm.at[cid, pl.ds(r0, ROWS_PER_TILE)])


def _sc_gather_scatter(table, gidx, sidx, zeros):
    return pl.kernel(
        _sc_body,
        out_type=jax.ShapeDtypeStruct((NC, ROWS, CH), jnp.float32),
        mesh=plsc.VectorSubcoreMesh(core_axis_name="c", subcore_axis_name="s"),
        scratch_types=[
            pltpu.VMEM((IDXBUF, CHUNK), jnp.int32),
            pltpu.VMEM((IDXBUF, CHUNK), jnp.int32),
            pltpu.VMEM((CHUNK, CH), jnp.float32),
            pltpu.VMEM((CHUNK, CH), jnp.float32),
            pltpu.VMEM_SHARED((ROWS, CH), jnp.float32),
            pltpu.SemaphoreType.DMA,
            pltpu.SemaphoreType.DMA,
        ],
    )(table, gidx, sidx, zeros)


def kernel(X, vertex, edges, W_lin, scale, eps):
    npad = NNZ_PAD - NNZ
    pad0 = jnp.zeros((npad,), jnp.int32)
    padt = jnp.full((npad,), TRASH, jnp.int32)
    nrow = NNZ_PAD // CHUNK
    vg = jnp.concatenate([vertex, pad0]).reshape(nrow, CHUNK)
    es = jnp.concatenate([edges, padt]).reshape(nrow, CHUNK)
    eg = jnp.concatenate([edges, pad0]).reshape(nrow, CHUNK)
    vs = jnp.concatenate([vertex, padt]).reshape(nrow, CHUNK)
    zeros = jnp.zeros((ROWS, CH), jnp.float32)

    escale = jnp.exp(scale).reshape(1, 1)
    X1 = _lorentz_tc(X, W_lin, escale)
    P = _sc_gather_scatter(X1, vg, es, zeros)
    Xe = _norm_tc(P)
    Q = _sc_gather_scatter(Xe, eg, vs, zeros)
    return _final_tc(Q, X1, eps.reshape(1, 1))


# speedup vs baseline: 4.0910x; 4.0910x over previous
import functools
import math

import jax
import jax.numpy as jnp
from jax import lax
from jax.experimental import pallas as pl
from jax.experimental.pallas import tpu as pltpu
from jax.experimental.pallas import tpu_sc as plsc

N_NODES = 10000
N_EDGES = 10000
NNZ = 320000
CH = 128

NC = 2
NS = 16
NW = NC * NS
CHUNK = 128
CHUNKS_PER_W = 80
PHS = (40, 40)
IDXBUF = 40
TOTAL_CHUNKS = NW * CHUNKS_PER_W
NNZ_PAD = TOTAL_CHUNKS * CHUNK
ROWS = 10112
ROWS_PER_TILE = ROWS // NS
TRASH = 10000


def _lorentz_body(x_ref, w_ref, esc_ref, o_ref):
    x = x_ref[...]
    w = w_ref[...]
    y = jnp.dot(x, w.T, preferred_element_type=jnp.float32)
    y0 = y[:, 0:1]
    time = jax.nn.sigmoid(y0) * esc_ref[0, 0] + 1.1
    sq = jnp.sum(y * y, axis=1, keepdims=True) - y0 * y0
    sq = jnp.clip(sq, 1e-8, None)
    s = (time * time - 1.0) / sq
    col = lax.broadcasted_iota(jnp.int32, y.shape, 1)
    o_ref[...] = jnp.where(col == 0, time, y * jnp.sqrt(s))


def _lorentz_tc(X, W, escale):
    grid = 10
    rows = N_NODES // grid
    return pl.pallas_call(
        _lorentz_body,
        grid=(grid,),
        in_specs=[
            pl.BlockSpec((rows, CH), lambda i: (i, 0)),
            pl.BlockSpec((CH, CH), lambda i: (0, 0)),
            pl.BlockSpec(memory_space=pltpu.SMEM),
        ],
        out_specs=pl.BlockSpec((rows, CH), lambda i: (i, 0)),
        out_shape=jax.ShapeDtypeStruct((N_NODES, CH), jnp.float32),
    )(X, W, escale)


def _norm_body(p_ref, o_ref):
    xe = p_ref[0] + p_ref[1]
    x0 = xe[:, 0:1]
    neg_inner = 2.0 * x0 * x0 - jnp.sum(xe * xe, axis=1, keepdims=True)
    denom = jnp.sqrt(jnp.clip(jnp.abs(neg_inner), 1e-8, None))
    o_ref[...] = xe / denom


def _norm_tc(P):
    grid = 4
    rows = ROWS // grid
    return pl.pallas_call(
        _norm_body,
        grid=(grid,),
        in_specs=[pl.BlockSpec((2, rows, CH), lambda i: (0, i, 0))],
        out_specs=pl.BlockSpec((rows, CH), lambda i: (i, 0)),
        out_shape=jax.ShapeDtypeStruct((ROWS, CH), jnp.float32),
    )(P)


def _final_body(q_ref, x1_ref, eps_ref, o_ref):
    o_ref[...] = eps_ref[0, 0] * (q_ref[0] + q_ref[1]) + x1_ref[...]


def _final_tc(Q, X1, eps):
    grid = 10
    rows = N_NODES // grid
    return pl.pallas_call(
        _final_body,
        grid=(grid,),
        in_specs=[
            pl.BlockSpec((2, rows, CH), lambda i: (0, i, 0)),
            pl.BlockSpec((rows, CH), lambda i: (i, 0)),
            pl.BlockSpec(memory_space=pltpu.SMEM),
        ],
        out_specs=pl.BlockSpec((rows, CH), lambda i: (i, 0)),
        out_shape=jax.ShapeDtypeStruct((N_NODES, CH), jnp.float32),
    )(Q, X1, eps)


def _sc_body(table_hbm, gidx_hbm, sidx_hbm, zeros_hbm, out_hbm,
             gidx_v, sidx_v, buf0, buf1, acc, sem0, sem1):
    cid = lax.axis_index("c")
    sid = lax.axis_index("s")
    r0 = sid * ROWS_PER_TILE
    pltpu.sync_copy(zeros_hbm.at[pl.ds(r0, ROWS_PER_TILE)],
                    acc.at[pl.ds(r0, ROWS_PER_TILE)])
    plsc.subcore_barrier()

    def run_chunks(first_chunk, phase_sizes):
        done = 0
        for pc in phase_sizes:
            pbase = first_chunk + done
            done += pc
            pltpu.sync_copy(gidx_hbm.at[pl.ds(pbase, pc)],
                            gidx_v.at[pl.ds(0, pc)])
            pltpu.sync_copy(sidx_hbm.at[pl.ds(pbase, pc)],
                            sidx_v.at[pl.ds(0, pc)])
            pltpu.async_copy(table_hbm.at[gidx_v.at[0]], buf0, sem0)

            def body(t, carry):
                j = 2 * t
                pltpu.async_copy(table_hbm.at[gidx_v.at[j + 1]], buf1, sem1)
                pltpu.make_async_copy(table_hbm.at[gidx_v.at[j]],
                                      buf0, sem0).wait()
                pltpu.sync_copy(buf0, acc.at[sidx_v.at[j]], add=True)

                @pl.when(j + 2 < pc)
                def _():
                    pltpu.async_copy(table_hbm.at[gidx_v.at[j + 2]],
                                     buf0, sem0)

                pltpu.make_async_copy(table_hbm.at[gidx_v.at[j + 1]],
                                      buf1, sem1).wait()
                pltpu.sync_copy(buf1, acc.at[sidx_v.at[j + 1]], add=True)
                return carry

            lax.fori_loop(0, pc // 2, body, 0)

    run_chunks((cid * NS + sid) * CHUNKS_PER_W, PHS)

    plsc.subcore_barrier()
    pltpu.sync_copy(acc.at[pl.ds(r0, ROWS_PER_TILE)],
                    out_hbm.at[cid, pl.ds(r0, ROWS_PER_TILE)])


def _sc_gather_scatter(table, gidx, sidx, zeros):
    return pl.kernel(
        _sc_body,
        out_type=jax.ShapeDtypeStruct((NC, ROWS, CH), jnp.float32),
        mesh=plsc.VectorSubcoreMesh(core_axis_name="c", subcore_axis_name="s"),
        scratch_types=[
            pltpu.VMEM((IDXBUF, CHUNK), jnp.int32),
            pltpu.VMEM((IDXBUF, CHUNK), jnp.int32),
            pltpu.VMEM((CHUNK, CH), jnp.float32),
            pltpu.VMEM((CHUNK, CH), jnp.float32),
            pltpu.VMEM_SHARED((ROWS, CH), jnp.float32),
            pltpu.SemaphoreType.DMA,
            pltpu.SemaphoreType.DMA,
        ],
    )(table, gidx, sidx, zeros)


def kernel(X, vertex, edges, W_lin, scale, eps):
    npad = NNZ_PAD - NNZ
    pad_i = jnp.arange(npad, dtype=jnp.int32)
    pad0 = pad_i % N_NODES
    padt = TRASH + pad_i % (ROWS - TRASH)
    nrow = NNZ_PAD // CHUNK
    vg = jnp.concatenate([vertex, pad0]).reshape(nrow, CHUNK)
    es = jnp.concatenate([edges, padt]).reshape(nrow, CHUNK)
    eg = jnp.concatenate([edges, pad0]).reshape(nrow, CHUNK)
    vs = jnp.concatenate([vertex, padt]).reshape(nrow, CHUNK)
    zeros = jnp.zeros((ROWS, CH), jnp.float32)

    escale = jnp.exp(scale).reshape(1, 1)
    X1 = _lorentz_tc(X, W_lin, escale)
    P = _sc_gather_scatter(X1, vg, es, zeros)
    Xe = _norm_tc(P)
    Q = _sc_gather_scatter(Xe, eg, vs, zeros)
    return _final_tc(Q, X1, eps.reshape(1, 1))


# speedup vs baseline: 4.1828x; 1.0224x over previous
import functools
import math

import jax
import jax.numpy as jnp
from jax import lax
from jax.experimental import pallas as pl
from jax.experimental.pallas import tpu as pltpu
from jax.experimental.pallas import tpu_sc as plsc

N_NODES = 10000
N_EDGES = 10000
NNZ = 320000
CH = 128

NC = 2
NS = 16
NW = NC * NS
CHUNK = 128
CHUNKS_PER_W = 80
PHASES = 2
PHASE_CHUNKS = CHUNKS_PER_W // PHASES
TOTAL_CHUNKS = NW * CHUNKS_PER_W
NNZ_PAD = TOTAL_CHUNKS * CHUNK
ROWS = 10112
ROWS_PER_TILE = ROWS // NS
TRASH = 10000


def _lorentz_body(x_ref, w_ref, esc_ref, o_ref):
    x = x_ref[...]
    w = w_ref[...]
    y = jnp.dot(x, w.T, preferred_element_type=jnp.float32)
    y0 = y[:, 0:1]
    time = jax.nn.sigmoid(y0) * esc_ref[0, 0] + 1.1
    sq = jnp.sum(y * y, axis=1, keepdims=True) - y0 * y0
    sq = jnp.clip(sq, 1e-8, None)
    s = (time * time - 1.0) / sq
    col = lax.broadcasted_iota(jnp.int32, y.shape, 1)
    o_ref[...] = jnp.where(col == 0, time, y * jnp.sqrt(s))


def _lorentz_tc(X, W, escale):
    grid = 10
    rows = N_NODES // grid
    return pl.pallas_call(
        _lorentz_body,
        grid=(grid,),
        in_specs=[
            pl.BlockSpec((rows, CH), lambda i: (i, 0)),
            pl.BlockSpec((CH, CH), lambda i: (0, 0)),
            pl.BlockSpec(memory_space=pltpu.SMEM),
        ],
        out_specs=pl.BlockSpec((rows, CH), lambda i: (i, 0)),
        out_shape=jax.ShapeDtypeStruct((N_NODES, CH), jnp.float32),
    )(X, W, escale)


def _norm_body(p_ref, o_ref):
    xe = p_ref[0] + p_ref[1]
    x0 = xe[:, 0:1]
    neg_inner = 2.0 * x0 * x0 - jnp.sum(xe * xe, axis=1, keepdims=True)
    denom = jnp.sqrt(jnp.clip(jnp.abs(neg_inner), 1e-8, None))
    o_ref[...] = xe / denom


def _norm_tc(P):
    grid = 4
    rows = ROWS // grid
    return pl.pallas_call(
        _norm_body,
        grid=(grid,),
        in_specs=[pl.BlockSpec((2, rows, CH), lambda i: (0, i, 0))],
        out_specs=pl.BlockSpec((rows, CH), lambda i: (i, 0)),
        out_shape=jax.ShapeDtypeStruct((ROWS, CH), jnp.float32),
    )(P)


def _final_body(q_ref, x1_ref, eps_ref, o_ref):
    o_ref[...] = eps_ref[0, 0] * (q_ref[0] + q_ref[1]) + x1_ref[...]


def _final_tc(Q, X1, eps):
    grid = 10
    rows = N_NODES // grid
    return pl.pallas_call(
        _final_body,
        grid=(grid,),
        in_specs=[
            pl.BlockSpec((2, rows, CH), lambda i: (0, i, 0)),
            pl.BlockSpec((rows, CH), lambda i: (i, 0)),
            pl.BlockSpec(memory_space=pltpu.SMEM),
        ],
        out_specs=pl.BlockSpec((rows, CH), lambda i: (i, 0)),
        out_shape=jax.ShapeDtypeStruct((N_NODES, CH), jnp.float32),
    )(Q, X1, eps)


def _sc_body(table_hbm, gidx_hbm, sidx_hbm, zeros_hbm, out_hbm,
             gidx_v, sidx_v, buf0, buf1, acc, sem0, sem1):
    cid = lax.axis_index("c")
    sid = lax.axis_index("s")
    base = (cid * NS + sid) * CHUNKS_PER_W
    pltpu.sync_copy(gidx_hbm.at[pl.ds(base, CHUNKS_PER_W)], gidx_v)
    r0 = sid * ROWS_PER_TILE
    pltpu.sync_copy(zeros_hbm.at[pl.ds(r0, ROWS_PER_TILE)],
                    acc.at[pl.ds(r0, ROWS_PER_TILE)])
    pltpu.async_copy(table_hbm.at[gidx_v.at[0]], buf0, sem0)
    plsc.subcore_barrier()

    for ph in range(PHASES):
        pltpu.sync_copy(
            sidx_hbm.at[pl.ds(base + ph * PHASE_CHUNKS, PHASE_CHUNKS)],
            sidx_v)

        def body(t, carry):
            j = ph * PHASE_CHUNKS + 2 * t
            s = 2 * t
            pltpu.async_copy(table_hbm.at[gidx_v.at[j + 1]], buf1, sem1)
            pltpu.make_async_copy(table_hbm.at[gidx_v.at[j]],
                                  buf0, sem0).wait()
            pltpu.sync_copy(buf0, acc.at[sidx_v.at[s]], add=True)

            @pl.when(j + 2 < CHUNKS_PER_W)
            def _():
                pltpu.async_copy(table_hbm.at[gidx_v.at[j + 2]], buf0, sem0)

            pltpu.make_async_copy(table_hbm.at[gidx_v.at[j + 1]],
                                  buf1, sem1).wait()
            pltpu.sync_copy(buf1, acc.at[sidx_v.at[s + 1]], add=True)
            return carry

        lax.fori_loop(0, PHASE_CHUNKS // 2, body, 0)

    plsc.subcore_barrier()
    pltpu.sync_copy(acc.at[pl.ds(r0, ROWS_PER_TILE)],
                    out_hbm.at[cid, pl.ds(r0, ROWS_PER_TILE)])


def _sc_gather_scatter(table, gidx, sidx, zeros):
    return pl.kernel(
        _sc_body,
        out_type=jax.ShapeDtypeStruct((NC, ROWS, CH), jnp.float32),
        mesh=plsc.VectorSubcoreMesh(core_axis_name="c", subcore_axis_name="s"),
        scratch_types=[
            pltpu.VMEM((CHUNKS_PER_W, CHUNK), jnp.int32),
            pltpu.VMEM((PHASE_CHUNKS, CHUNK), jnp.int32),
            pltpu.VMEM((CHUNK, CH), jnp.float32),
            pltpu.VMEM((CHUNK, CH), jnp.float32),
            pltpu.VMEM_SHARED((ROWS, CH), jnp.float32),
            pltpu.SemaphoreType.DMA,
            pltpu.SemaphoreType.DMA,
        ],
    )(table, gidx, sidx, zeros)


def kernel(X, vertex, edges, W_lin, scale, eps):
    npad = NNZ_PAD - NNZ
    pad_i = jnp.arange(npad, dtype=jnp.int32)
    pad0 = pad_i % N_NODES
    padt = TRASH + pad_i % (ROWS - TRASH)
    nrow = NNZ_PAD // CHUNK
    vg = jnp.concatenate([vertex, pad0]).reshape(nrow, CHUNK)
    es = jnp.concatenate([edges, padt]).reshape(nrow, CHUNK)
    eg = jnp.concatenate([edges, pad0]).reshape(nrow, CHUNK)
    vs = jnp.concatenate([vertex, padt]).reshape(nrow, CHUNK)
    zeros = jnp.zeros((ROWS, CH), jnp.float32)

    escale = jnp.exp(scale).reshape(1, 1)
    X1 = _lorentz_tc(X, W_lin, escale)
    P = _sc_gather_scatter(X1, vg, es, zeros)
    Xe = _norm_tc(P)
    Q = _sc_gather_scatter(Xe, eg, vs, zeros)
    return _final_tc(Q, X1, eps.reshape(1, 1))


# speedup vs baseline: 4.1965x; 1.0033x over previous
import jax
import jax.numpy as jnp
from jax import lax
from jax.experimental import pallas as pl
from jax.experimental.pallas import tpu as pltpu
from jax.experimental.pallas import tpu_sc as plsc

N_NODES = 10000
N_EDGES = 10000
NNZ = 320000
CH = 128

NC = 2
NS = 16
NW = NC * NS
CHUNK = 128
CHUNKS_PER_W = 80
PHASES = 2
PHASE_CHUNKS = CHUNKS_PER_W // PHASES
TOTAL_CHUNKS = NW * CHUNKS_PER_W
NNZ_PAD = TOTAL_CHUNKS * CHUNK
ROWS = 10112
ROWS_PER_TILE = ROWS // NS
TRASH = 10000


def _lorentz_body(x_ref, w_ref, esc_ref, o_ref):
    x = x_ref[...]
    w = w_ref[...]
    y = jnp.dot(x, w.T, preferred_element_type=jnp.float32)
    y0 = y[:, 0:1]
    time = jax.nn.sigmoid(y0) * esc_ref[0, 0] + 1.1
    sq = jnp.sum(y * y, axis=1, keepdims=True) - y0 * y0
    sq = jnp.clip(sq, 1e-8, None)
    s = (time * time - 1.0) / sq
    col = lax.broadcasted_iota(jnp.int32, y.shape, 1)
    o_ref[...] = jnp.where(col == 0, time, y * jnp.sqrt(s))


def _lorentz_tc(X, W, escale):
    grid = 10
    rows = N_NODES // grid
    return pl.pallas_call(
        _lorentz_body,
        grid=(grid,),
        in_specs=[
            pl.BlockSpec((rows, CH), lambda i: (i, 0)),
            pl.BlockSpec((CH, CH), lambda i: (0, 0)),
            pl.BlockSpec(memory_space=pltpu.SMEM),
        ],
        out_specs=pl.BlockSpec((rows, CH), lambda i: (i, 0)),
        out_shape=jax.ShapeDtypeStruct((N_NODES, CH), jnp.float32),
    )(X, W, escale)


def _norm_body(p_ref, o_ref):
    xe = p_ref[0] + p_ref[1]
    x0 = xe[:, 0:1]
    neg_inner = 2.0 * x0 * x0 - jnp.sum(xe * xe, axis=1, keepdims=True)
    denom = jnp.sqrt(jnp.clip(jnp.abs(neg_inner), 1e-8, None))
    o_ref[...] = xe / denom


def _norm_tc(P):
    grid = 4
    rows = ROWS // grid
    return pl.pallas_call(
        _norm_body,
        grid=(grid,),
        in_specs=[pl.BlockSpec((2, rows, CH), lambda i: (0, i, 0))],
        out_specs=pl.BlockSpec((rows, CH), lambda i: (i, 0)),
        out_shape=jax.ShapeDtypeStruct((ROWS, CH), jnp.float32),
    )(P)


def _final_body(q_ref, x1_ref, eps_ref, o_ref):
    o_ref[...] = eps_ref[0, 0] * (q_ref[0] + q_ref[1]) + x1_ref[...]


def _final_tc(Q, X1, eps):
    grid = 10
    rows = N_NODES // grid
    return pl.pallas_call(
        _final_body,
        grid=(grid,),
        in_specs=[
            pl.BlockSpec((2, rows, CH), lambda i: (0, i, 0)),
            pl.BlockSpec((rows, CH), lambda i: (i, 0)),
            pl.BlockSpec(memory_space=pltpu.SMEM),
        ],
        out_specs=pl.BlockSpec((rows, CH), lambda i: (i, 0)),
        out_shape=jax.ShapeDtypeStruct((N_NODES, CH), jnp.float32),
    )(Q, X1, eps)


def _sc_body(table_hbm, gidx_hbm, sidx_hbm, zeros_hbm, out_hbm,
             gidx_v, sidx_v, buf0, buf1, acc, sem0, sem1):
    cid = lax.axis_index("c")
    sid = lax.axis_index("s")
    base = (cid * NS + sid) * CHUNKS_PER_W
    pltpu.sync_copy(gidx_hbm.at[pl.ds(base, CHUNKS_PER_W)], gidx_v)
    r0 = sid * ROWS_PER_TILE
    pltpu.sync_copy(zeros_hbm.at[pl.ds(r0, ROWS_PER_TILE)],
                    acc.at[pl.ds(r0, ROWS_PER_TILE)])
    pltpu.async_copy(table_hbm.at[gidx_v.at[0]], buf0, sem0)
    plsc.subcore_barrier()

    for ph in range(PHASES):
        pltpu.sync_copy(
            sidx_hbm.at[pl.ds(base + ph * PHASE_CHUNKS, PHASE_CHUNKS)],
            sidx_v)

        def body(t, carry):
            j = ph * PHASE_CHUNKS + 2 * t
            s = 2 * t
            pltpu.async_copy(table_hbm.at[gidx_v.at[j + 1]], buf1, sem1)
            pltpu.make_async_copy(table_hbm.at[gidx_v.at[j]],
                                  buf0, sem0).wait()
            pltpu.sync_copy(buf0, acc.at[sidx_v.at[s]], add=True)

            @pl.when(j + 2 < CHUNKS_PER_W)
            def _():
                pltpu.async_copy(table_hbm.at[gidx_v.at[j + 2]], buf0, sem0)

            pltpu.make_async_copy(table_hbm.at[gidx_v.at[j + 1]],
                                  buf1, sem1).wait()
            pltpu.sync_copy(buf1, acc.at[sidx_v.at[s + 1]], add=True)
            return carry

        lax.fori_loop(0, PHASE_CHUNKS // 2, body, 0)

    plsc.subcore_barrier()
    pltpu.sync_copy(acc.at[pl.ds(r0, ROWS_PER_TILE)],
                    out_hbm.at[cid, pl.ds(r0, ROWS_PER_TILE)])


def _sc_gather_scatter(table, gidx, sidx, zeros):
    return pl.kernel(
        _sc_body,
        out_type=jax.ShapeDtypeStruct((NC, ROWS, CH), jnp.float32),
        mesh=plsc.VectorSubcoreMesh(core_axis_name="c", subcore_axis_name="s"),
        scratch_types=[
            pltpu.VMEM((CHUNKS_PER_W, CHUNK), jnp.int32),
            pltpu.VMEM((PHASE_CHUNKS, CHUNK), jnp.int32),
            pltpu.VMEM((CHUNK, CH), jnp.float32),
            pltpu.VMEM((CHUNK, CH), jnp.float32),
            pltpu.VMEM_SHARED((ROWS, CH), jnp.float32),
            pltpu.SemaphoreType.DMA,
            pltpu.SemaphoreType.DMA,
        ],
    )(table, gidx, sidx, zeros)


def kernel(X, vertex, edges, W_lin, scale, eps):
    npad = NNZ_PAD - NNZ
    pad_i = jnp.arange(npad, dtype=jnp.int32)
    pad0 = pad_i % N_NODES
    padt = TRASH + pad_i % (ROWS - TRASH)
    nrow = NNZ_PAD // CHUNK
    vg = jnp.concatenate([vertex, pad0]).reshape(nrow, CHUNK)
    es = jnp.concatenate([edges, padt]).reshape(nrow, CHUNK)
    eg = jnp.concatenate([edges, pad0]).reshape(nrow, CHUNK)
    vs = jnp.concatenate([vertex, padt]).reshape(nrow, CHUNK)
    zeros = jnp.zeros((ROWS, CH), jnp.float32)

    escale = jnp.exp(scale).reshape(1, 1)
    X1 = _lorentz_tc(X, W_lin, escale)
    P = _sc_gather_scatter(X1, vg, es, zeros)
    Xe = _norm_tc(P)
    Q = _sc_gather_scatter(Xe, eg, vs, zeros)
    return _final_tc(Q, X1, eps.reshape(1, 1))
